# Initial kernel scaffold; baseline (speedup 1.0000x reference)
import jax
import jax.numpy as jnp
from jax.experimental import pallas as pl


def kernel(users, items, user_emb, item_emb, adj_src, adj_dst, adj_val, hp_src, hp_dst, hp_val, W, b):
    raise NotImplementedError("write your pallas kernel here")


# speedup vs baseline: 3.4695x; 3.4695x over previous
import functools

import jax
import jax.numpy as jnp
from jax import lax
from jax.experimental import pallas as pl
from jax.experimental.pallas import tpu as pltpu
from jax.experimental.pallas import tpu_sc as plsc

_NC = 2
_NS = 16
_C = 128


def _linear_body(x_ref, w_ref, yp_ref, yd_ref):
    x = jnp.concatenate([x_ref[0], x_ref[1]], axis=1)
    w = w_ref[...]
    yp = jnp.dot(x, w[:32], preferred_element_type=jnp.float32)
    yd = jnp.dot(x, w[32:], preferred_element_type=jnp.float32)
    yp_ref[0], yp_ref[1] = yp[:, :16], yp[:, 16:]
    yd_ref[0], yd_ref[1] = yd[:, :16], yd[:, 16:]


def _linear(x2, w):
    n = x2.shape[1]
    bn = 2048
    spec = pl.BlockSpec((2, bn, 16), lambda i: (0, i, 0))
    wspec = pl.BlockSpec((64, 32), lambda i: (0, 0))
    out = jax.ShapeDtypeStruct((2, n, 16), jnp.float32)
    return pl.pallas_call(
        _linear_body,
        grid=((n + bn - 1) // bn,),
        in_specs=[spec, wspec],
        out_specs=[spec, spec],
        out_shape=[out, out],
    )(x2, w)


def _spmm_layer(npad, epad):
    ept = epad // _NS
    nchunks = ept // _C
    rpt = npad // _NS
    fb_full = rpt // _C
    fb_rem = rpt - fb_full * _C
    mesh = plsc.VectorSubcoreMesh(core_axis_name="c", subcore_axis_name="s")

    @functools.partial(
        pl.kernel,
        out_type=jax.ShapeDtypeStruct((2 * npad, 16), jnp.float32),
        mesh=mesh,
        compiler_params=pltpu.CompilerParams(use_tc_tiling_on_sc=False),
        scratch_types=[
            pltpu.VMEM_SHARED((npad, 16), jnp.float32),
            pltpu.VMEM((_C,), jnp.int32),
            pltpu.VMEM((_C,), jnp.int32),
            pltpu.VMEM((_C,), jnp.float32),
            pltpu.VMEM((_C, 16), jnp.float32),
            pltpu.VMEM((_C, 16), jnp.float32),
            pltpu.SemaphoreType.DMA,
        ],
    )
    def k(yp_hbm, yd_hbm, asrc, adst, aval, hsrc, hdst, hval, b_hbm, out_hbm,
          acc, srcb, dstb, valb, rowsb, fillb, sem):
        c = lax.axis_index("c")
        s = lax.axis_index("s")
        cn = c * npad

        pltpu.sync_copy(b_hbm.at[pl.ds(c * 16, 16)], fillb.at[0])
        bv = fillb[0, :]
        for r in range(1, _C):
            fillb[r, :] = bv
        ibase = s * rpt
        for j in range(fb_full):
            pltpu.sync_copy(fillb, acc.at[pl.ds(ibase + j * _C, _C)])
        if fb_rem:
            pltpu.sync_copy(fillb.at[pl.ds(0, fb_rem)],
                            acc.at[pl.ds(ibase + fb_full * _C, fb_rem)])
        plsc.subcore_barrier()

        def edge_pass(src_hbm, dst_hbm, val_hbm, y_hbm):
            ebase = s * ept

            @pl.loop(0, nchunks)
            def _(i):
                base = ebase + i * _C
                pltpu.sync_copy(src_hbm.at[pl.ds(base, _C)], srcb)
                pltpu.sync_copy(dst_hbm.at[pl.ds(base, _C)], dstb)
                pltpu.sync_copy(val_hbm.at[pl.ds(base, _C)], valb)
                for g in range(_C // 16):
                    sl = pl.ds(g * 16, 16)
                    srcb[sl] = srcb[sl] + cn
                pltpu.async_copy(y_hbm.at[srcb], rowsb, sem).wait()
                for g in range(_C // 16):
                    vv = valb[pl.ds(g * 16, 16)]
                    for j in range(16):
                        e = g * 16 + j
                        rowsb[e, :] = rowsb[e, :] * vv[j]
                pltpu.sync_copy(rowsb, acc.at[dstb], add=True)

        edge_pass(asrc, adst, aval, yp_hbm)
        edge_pass(hsrc, hdst, hval, yd_hbm)
        plsc.subcore_barrier()

        pltpu.sync_copy(acc.at[pl.ds(ibase, rpt)],
                        out_hbm.at[pl.ds(cn + ibase, rpt)])

    return k


def _gather_out(npad, u_count, batch):
    per_tile = batch // _NS
    mesh = plsc.VectorSubcoreMesh(core_axis_name="c", subcore_axis_name="s")
    out = jax.ShapeDtypeStruct((2 * batch, 16), jnp.float32)

    @functools.partial(
        pl.kernel,
        out_type=(out, out),
        mesh=mesh,
        compiler_params=pltpu.CompilerParams(use_tc_tiling_on_sc=False),
        scratch_types=[
            pltpu.VMEM((_C,), jnp.int32),
            pltpu.VMEM((_C, 16), jnp.float32),
            pltpu.SemaphoreType.DMA,
        ],
    )
    def k(x_hbm, u_hbm, i_hbm, uo_hbm, io_hbm, idxb, rowsb, sem):
        c = lax.axis_index("c")
        s = lax.axis_index("s")
        for idx_hbm, off, out_hbm in ((u_hbm, c * npad, uo_hbm),
                                      (i_hbm, c * npad + u_count, io_hbm)):
            for ch in range(per_tile // _C):
                base = s * per_tile + ch * _C
                pltpu.sync_copy(idx_hbm.at[pl.ds(base, _C)], idxb)
                for g in range(_C // 16):
                    sl = pl.ds(g * 16, 16)
                    idxb[sl] = idxb[sl] + off
                pltpu.async_copy(x_hbm.at[idxb], rowsb, sem).wait()
                pltpu.sync_copy(rowsb, out_hbm.at[pl.ds(c * batch + base, _C)])

    return k


def kernel(users, items, user_emb, item_emb, adj_src, adj_dst, adj_val,
           hp_src, hp_dst, hp_val, W, b):
    u_count, e = user_emb.shape
    n = u_count + item_emb.shape[0]
    npad = ((n + 127) // 128) * 128
    nlayers = W.shape[0]
    batch = users.shape[0]

    ego = jnp.concatenate([user_emb, item_emb], axis=0)
    ego = jnp.pad(ego, ((0, npad - n), (0, 0)))
    x2 = jnp.stack([ego[:, : e // 2], ego[:, e // 2:]], axis=0)

    nnz = adj_src.shape[0]
    quant = _NS * _C
    epad = ((nnz + quant - 1) // quant) * quant
    pad = epad - nnz
    a_src = jnp.pad(adj_src, (0, pad))
    a_dst = jnp.pad(adj_dst, (0, pad))
    a_val = jnp.pad(adj_val, (0, pad))
    h_src = jnp.pad(hp_src, (0, pad))
    h_dst = jnp.pad(hp_dst, (0, pad))
    h_val = jnp.pad(hp_val, (0, pad))

    spmm = _spmm_layer(npad, epad)
    x_flat = x2.reshape(2 * npad, 16)
    for k in range(nlayers):
        yp2, yd2 = _linear(x2, W[k])
        x_flat = spmm(yp2.reshape(2 * npad, 16), yd2.reshape(2 * npad, 16),
                      a_src, a_dst, a_val, h_src, h_dst, h_val, b[k])
        x2 = x_flat.reshape(2, npad, 16)

    uo, io = _gather_out(npad, u_count, batch)(x_flat, users, items)
    u2 = uo.reshape(2, batch, 16)
    i2 = io.reshape(2, batch, 16)
    user_embeddings = jnp.concatenate([u2[0], u2[1]], axis=1)
    item_embeddings = jnp.concatenate([i2[0], i2[1]], axis=1)
    return (user_embeddings, item_embeddings)


# speedup vs baseline: 8.7749x; 2.5292x over previous
import functools

import jax
import jax.numpy as jnp
from jax import lax
from jax.experimental import pallas as pl
from jax.experimental.pallas import tpu as pltpu
from jax.experimental.pallas import tpu_sc as plsc

_NC = 2
_NS = 16
_C = 128


def _linear_body(x_ref, w_ref, yp_ref, yd_ref):
    x = jnp.concatenate([x_ref[0], x_ref[1]], axis=1)
    w = w_ref[...]
    yp = jnp.dot(x, w[:32], preferred_element_type=jnp.float32)
    yd = jnp.dot(x, w[32:], preferred_element_type=jnp.float32)
    yp_ref[0], yp_ref[1] = yp[:, :16], yp[:, 16:]
    yd_ref[0], yd_ref[1] = yd[:, :16], yd[:, 16:]


def _linear(x2, w):
    n = x2.shape[1]
    bn = 2048
    spec = pl.BlockSpec((2, bn, 16), lambda i: (0, i, 0))
    wspec = pl.BlockSpec((64, 32), lambda i: (0, 0))
    out = jax.ShapeDtypeStruct((2, n, 16), jnp.float32)
    return pl.pallas_call(
        _linear_body,
        grid=((n + bn - 1) // bn,),
        in_specs=[spec, wspec],
        out_specs=[spec, spec],
        out_shape=[out, out],
    )(x2, w)


_RPB = 8


def _spmm_layer(npad, epad):
    ept = epad // _NS
    nbatch = ept // (_C * _RPB)
    crpt = ept // _C
    rpt = npad // _NS
    fb_full = rpt // _C
    fb_rem = rpt - fb_full * _C
    mesh = plsc.VectorSubcoreMesh(core_axis_name="c", subcore_axis_name="s")

    @functools.partial(
        pl.kernel,
        out_type=jax.ShapeDtypeStruct((2 * npad, 16), jnp.float32),
        mesh=mesh,
        compiler_params=pltpu.CompilerParams(use_tc_tiling_on_sc=False),
        scratch_types=[
            pltpu.VMEM_SHARED((npad, 16), jnp.float32),
            pltpu.VMEM((_RPB, _C), jnp.int32),
            pltpu.VMEM((_RPB, _C), jnp.int32),
            pltpu.VMEM((_RPB, _C), jnp.int32),
            pltpu.VMEM((_RPB, _C), jnp.int32),
            pltpu.VMEM((_RPB, _C), jnp.float32),
            pltpu.VMEM((_RPB, _C), jnp.float32),
            pltpu.VMEM((_C, 16), jnp.float32),
            pltpu.VMEM((_C, 16), jnp.float32),
            pltpu.VMEM((_C, 16), jnp.float32),
            pltpu.SemaphoreType.DMA,
            pltpu.SemaphoreType.DMA,
            pltpu.SemaphoreType.DMA,
            pltpu.SemaphoreType.DMA,
            pltpu.SemaphoreType.DMA,
            pltpu.SemaphoreType.DMA,
        ],
    )
    def k(yp_hbm, yd_hbm, asrc, adst, aval, hsrc, hdst, hval, b_hbm, out_hbm,
          acc, src0, src1, dst0, dst1, val0, val1, rows0, rows1, fillb,
          semi0, semi1, semg0, semg1, sems0, sems1):
        c = lax.axis_index("c")
        s = lax.axis_index("s")
        cn = c * npad
        srcB, dstB, valB = (src0, src1), (dst0, dst1), (val0, val1)
        rows, semI = (rows0, rows1), (semi0, semi1)
        semG, semS = (semg0, semg1), (sems0, sems1)

        pltpu.sync_copy(b_hbm.at[pl.ds(c * 16, 16)], fillb.at[0])
        bv = fillb[0, :]
        for r in range(1, _C):
            fillb[r, :] = bv
        ibase = s * rpt
        for j in range(fb_full):
            pltpu.sync_copy(fillb, acc.at[pl.ds(ibase + j * _C, _C)])
        if fb_rem:
            pltpu.sync_copy(fillb.at[pl.ds(0, fb_rem)],
                            acc.at[pl.ds(ibase + fb_full * _C, fb_rem)])
        plsc.subcore_barrier()

        def edge_pass(src2, dst2, val2, y_hbm):
            rbase = s * crpt

            def idx_descs(bi, kk):
                ro = rbase + bi * _RPB
                return (
                    pltpu.make_async_copy(src2.at[pl.ds(ro, _RPB)], srcB[kk], semI[kk]),
                    pltpu.make_async_copy(dst2.at[pl.ds(ro, _RPB)], dstB[kk], semI[kk]),
                    pltpu.make_async_copy(val2.at[pl.ds(ro, _RPB)], valB[kk], semI[kk]),
                )

            for d in idx_descs(0, 0):
                d.start()

            @pl.loop(0, nbatch, step=2)
            def _(bi):
                for kk in (0, 1):
                    b = bi + kk
                    for d in idx_descs(b, kk):
                        d.wait()

                    @pl.when(b + 1 < nbatch)
                    def _():
                        for d in idx_descs(b + 1, kk ^ 1):
                            d.start()

                    for r in range(_RPB):
                        for g in range(_C // 16):
                            sl = pl.ds(g * 16, 16)
                            srcB[kk][r, sl] = srcB[kk][r, sl] + cn

                    gd = {}
                    sd = {}
                    gd[0] = pltpu.async_copy(y_hbm.at[srcB[kk].at[0]],
                                             rows[0], semG[0])
                    for ch in range(_RPB):
                        cur = ch & 1
                        gd[ch].wait()
                        if ch + 1 < _RPB:
                            if ch >= 1:
                                sd[ch - 1].wait()
                            gd[ch + 1] = pltpu.async_copy(
                                y_hbm.at[srcB[kk].at[ch + 1]],
                                rows[cur ^ 1], semG[cur ^ 1])
                        for g in range(_C // 16):
                            vv = valB[kk][ch, pl.ds(g * 16, 16)]
                            for j in range(16):
                                e = g * 16 + j
                                rows[cur][e, :] = rows[cur][e, :] * vv[j]
                        sd[ch] = pltpu.make_async_copy(
                            rows[cur], acc.at[dstB[kk].at[ch]], semS[cur])
                        sd[ch].start(add=True)
                    sd[_RPB - 2].wait()
                    sd[_RPB - 1].wait()

        edge_pass(asrc, adst, aval, yp_hbm)
        edge_pass(hsrc, hdst, hval, yd_hbm)
        plsc.subcore_barrier()

        pltpu.sync_copy(acc.at[pl.ds(ibase, rpt)],
                        out_hbm.at[pl.ds(cn + ibase, rpt)])

    return k


def _gather_out(npad, u_count, batch):
    per_tile = batch // _NS
    mesh = plsc.VectorSubcoreMesh(core_axis_name="c", subcore_axis_name="s")
    out = jax.ShapeDtypeStruct((2 * batch, 16), jnp.float32)

    @functools.partial(
        pl.kernel,
        out_type=(out, out),
        mesh=mesh,
        compiler_params=pltpu.CompilerParams(use_tc_tiling_on_sc=False),
        scratch_types=[
            pltpu.VMEM((_C,), jnp.int32),
            pltpu.VMEM((_C, 16), jnp.float32),
            pltpu.SemaphoreType.DMA,
        ],
    )
    def k(x_hbm, u_hbm, i_hbm, uo_hbm, io_hbm, idxb, rowsb, sem):
        c = lax.axis_index("c")
        s = lax.axis_index("s")
        for idx_hbm, off, out_hbm in ((u_hbm, c * npad, uo_hbm),
                                      (i_hbm, c * npad + u_count, io_hbm)):
            for ch in range(per_tile // _C):
                base = s * per_tile + ch * _C
                pltpu.sync_copy(idx_hbm.at[pl.ds(base, _C)], idxb)
                for g in range(_C // 16):
                    sl = pl.ds(g * 16, 16)
                    idxb[sl] = idxb[sl] + off
                pltpu.async_copy(x_hbm.at[idxb], rowsb, sem).wait()
                pltpu.sync_copy(rowsb, out_hbm.at[pl.ds(c * batch + base, _C)])

    return k


def kernel(users, items, user_emb, item_emb, adj_src, adj_dst, adj_val,
           hp_src, hp_dst, hp_val, W, b):
    u_count, e = user_emb.shape
    n = u_count + item_emb.shape[0]
    npad = ((n + 127) // 128) * 128
    nlayers = W.shape[0]
    batch = users.shape[0]

    ego = jnp.concatenate([user_emb, item_emb], axis=0)
    ego = jnp.pad(ego, ((0, npad - n), (0, 0)))
    x2 = jnp.stack([ego[:, : e // 2], ego[:, e // 2:]], axis=0)

    nnz = adj_src.shape[0]
    quant = _NS * _C * _RPB
    epad = ((nnz + quant - 1) // quant) * quant
    pad = epad - nnz
    a_src = jnp.pad(adj_src, (0, pad)).reshape(-1, _C)
    a_dst = jnp.pad(adj_dst, (0, pad)).reshape(-1, _C)
    a_val = jnp.pad(adj_val, (0, pad)).reshape(-1, _C)
    h_src = jnp.pad(hp_src, (0, pad)).reshape(-1, _C)
    h_dst = jnp.pad(hp_dst, (0, pad)).reshape(-1, _C)
    h_val = jnp.pad(hp_val, (0, pad)).reshape(-1, _C)

    spmm = _spmm_layer(npad, epad)
    x_flat = x2.reshape(2 * npad, 16)
    for k in range(nlayers):
        yp2, yd2 = _linear(x2, W[k])
        x_flat = spmm(yp2.reshape(2 * npad, 16), yd2.reshape(2 * npad, 16),
                      a_src, a_dst, a_val, h_src, h_dst, h_val, b[k])
        x2 = x_flat.reshape(2, npad, 16)

    uo, io = _gather_out(npad, u_count, batch)(x_flat, users, items)
    u2 = uo.reshape(2, batch, 16)
    i2 = io.reshape(2, batch, 16)
    user_embeddings = jnp.concatenate([u2[0], u2[1]], axis=1)
    item_embeddings = jnp.concatenate([i2[0], i2[1]], axis=1)
    return (user_embeddings, item_embeddings)


# speedup vs baseline: 11.6166x; 1.3238x over previous
import functools

import jax
import jax.numpy as jnp
from jax import lax
from jax.experimental import pallas as pl
from jax.experimental.pallas import tpu as pltpu
from jax.experimental.pallas import tpu_sc as plsc

_NC = 2
_NS = 16
_C = 128


def _linear_body(x_ref, w_ref, yp_ref, yd_ref):
    x = jnp.concatenate([x_ref[0], x_ref[1]], axis=1)
    w = w_ref[...]
    yp = jnp.dot(x, w[:32], preferred_element_type=jnp.float32)
    yd = jnp.dot(x, w[32:], preferred_element_type=jnp.float32)
    yp_ref[0], yp_ref[1] = yp[:, :16], yp[:, 16:]
    yd_ref[0], yd_ref[1] = yd[:, :16], yd[:, 16:]


def _linear(x2, w):
    n = x2.shape[1]
    bn = 2048
    spec = pl.BlockSpec((2, bn, 16), lambda i: (0, i, 0))
    wspec = pl.BlockSpec((64, 32), lambda i: (0, 0))
    out = jax.ShapeDtypeStruct((2, n, 16), jnp.float32)
    return pl.pallas_call(
        _linear_body,
        grid=((n + bn - 1) // bn,),
        in_specs=[spec, wspec],
        out_specs=[spec, spec],
        out_shape=[out, out],
    )(x2, w)


_RPB = 16
_NB = 6
_KA = 4


def _spmm_layer(npad, epad):
    ept = epad // _NS
    nbatch = ept // (_C * _RPB)
    crpt = ept // _C
    rpt = npad // _NS
    fb_full = rpt // _C
    fb_rem = rpt - fb_full * _C
    mesh = plsc.VectorSubcoreMesh(core_axis_name="c", subcore_axis_name="s")

    @functools.partial(
        pl.kernel,
        out_type=jax.ShapeDtypeStruct((2 * npad, 16), jnp.float32),
        mesh=mesh,
        compiler_params=pltpu.CompilerParams(use_tc_tiling_on_sc=False),
        scratch_types=(
            [pltpu.VMEM_SHARED((npad, 16), jnp.float32)]
            + [pltpu.VMEM((_RPB, _C), jnp.int32)] * 4
            + [pltpu.VMEM((_RPB, _C), jnp.float32)] * 2
            + [pltpu.VMEM((_C, 16), jnp.float32)] * _NB
            + [pltpu.VMEM((_C, 16), jnp.float32)]
            + [pltpu.SemaphoreType.DMA] * (2 + 2 * _NB)
        ),
    )
    def k(yp_hbm, yd_hbm, asrc, adst, aval, hsrc, hdst, hval, b_hbm, out_hbm,
          acc, src0, src1, dst0, dst1, val0, val1, *rest):
        rows = rest[:_NB]
        fillb = rest[_NB]
        semI = rest[_NB + 1:_NB + 3]
        semG = rest[_NB + 3:_NB + 3 + _NB]
        semS = rest[_NB + 3 + _NB:_NB + 3 + 2 * _NB]
        c = lax.axis_index("c")
        s = lax.axis_index("s")
        cn = c * npad
        srcB, dstB, valB = (src0, src1), (dst0, dst1), (val0, val1)

        pltpu.sync_copy(b_hbm.at[pl.ds(c * 16, 16)], fillb.at[0])
        bv = fillb[0, :]
        for r in range(1, _C):
            fillb[r, :] = bv
        ibase = s * rpt
        for j in range(fb_full):
            pltpu.sync_copy(fillb, acc.at[pl.ds(ibase + j * _C, _C)])
        if fb_rem:
            pltpu.sync_copy(fillb.at[pl.ds(0, fb_rem)],
                            acc.at[pl.ds(ibase + fb_full * _C, fb_rem)])
        plsc.subcore_barrier()

        def scat_wait(t):
            pltpu.make_async_copy(rows[t % _NB], acc.at[dstB[0].at[0]],
                                  semS[t % _NB]).wait()

        def edge_pass(src2, dst2, val2, y_hbm):
            rbase = s * crpt

            def idx_descs(bi, kk):
                ro = rbase + bi * _RPB
                return (
                    pltpu.make_async_copy(src2.at[pl.ds(ro, _RPB)], srcB[kk], semI[kk]),
                    pltpu.make_async_copy(dst2.at[pl.ds(ro, _RPB)], dstB[kk], semI[kk]),
                    pltpu.make_async_copy(val2.at[pl.ds(ro, _RPB)], valB[kk], semI[kk]),
                )

            for d in idx_descs(0, 0):
                d.start()

            @pl.loop(0, nbatch, step=2)
            def _(bi):
                for kk in (0, 1):
                    b = bi + kk
                    for d in idx_descs(b, kk):
                        d.wait()

                    @pl.when(b + 1 < nbatch)
                    def _():
                        for d in idx_descs(b + 1, kk ^ 1):
                            d.start()

                    @pl.loop(0, _RPB)
                    def _(r):
                        for g in range(_C // 16):
                            sl = pl.ds(g * 16, 16)
                            srcB[kk][r, sl] = srcB[kk][r, sl] + cn

                    def gissue(t):
                        return pltpu.async_copy(y_hbm.at[srcB[kk].at[t]],
                                                rows[t % _NB], semG[t % _NB])

                    gd = {}
                    sd = {}
                    for t in range(_KA):

                        @pl.when(b > 0)
                        def _(t=t):
                            scat_wait(t)

                        gd[t] = gissue(t)

                    for ch in range(_RPB):
                        gd[ch].wait()
                        t = ch + _KA
                        if t < _RPB:
                            if t < _NB:

                                @pl.when(b > 0)
                                def _(t=t):
                                    scat_wait(t)

                            else:
                                sd[t - _NB].wait()
                            gd[t] = gissue(t)
                        rb = rows[ch % _NB]

                        @pl.loop(0, _C // 16)
                        def _(g, ch=ch, rb=rb):
                            gb = g * 16
                            vv = valB[kk][ch, pl.ds(gb, 16)]
                            for j in range(16):
                                rb[gb + j, :] = rb[gb + j, :] * vv[j]
                        sd[ch] = pltpu.make_async_copy(
                            rows[ch % _NB], acc.at[dstB[kk].at[ch]],
                            semS[ch % _NB])
                        sd[ch].start(add=True)

            for t in range(_RPB - _NB, _RPB):
                scat_wait(t)

        edge_pass(asrc, adst, aval, yp_hbm)
        edge_pass(hsrc, hdst, hval, yd_hbm)
        plsc.subcore_barrier()

        pltpu.sync_copy(acc.at[pl.ds(ibase, rpt)],
                        out_hbm.at[pl.ds(cn + ibase, rpt)])

    return k


def _gather_out(npad, u_count, batch):
    per_tile = batch // _NS
    mesh = plsc.VectorSubcoreMesh(core_axis_name="c", subcore_axis_name="s")
    out = jax.ShapeDtypeStruct((2 * batch, 16), jnp.float32)

    @functools.partial(
        pl.kernel,
        out_type=(out, out),
        mesh=mesh,
        compiler_params=pltpu.CompilerParams(use_tc_tiling_on_sc=False),
        scratch_types=[
            pltpu.VMEM((_C,), jnp.int32),
            pltpu.VMEM((_C, 16), jnp.float32),
            pltpu.SemaphoreType.DMA,
        ],
    )
    def k(x_hbm, u_hbm, i_hbm, uo_hbm, io_hbm, idxb, rowsb, sem):
        c = lax.axis_index("c")
        s = lax.axis_index("s")
        for idx_hbm, off, out_hbm in ((u_hbm, c * npad, uo_hbm),
                                      (i_hbm, c * npad + u_count, io_hbm)):
            for ch in range(per_tile // _C):
                base = s * per_tile + ch * _C
                pltpu.sync_copy(idx_hbm.at[pl.ds(base, _C)], idxb)
                for g in range(_C // 16):
                    sl = pl.ds(g * 16, 16)
                    idxb[sl] = idxb[sl] + off
                pltpu.async_copy(x_hbm.at[idxb], rowsb, sem).wait()
                pltpu.sync_copy(rowsb, out_hbm.at[pl.ds(c * batch + base, _C)])

    return k


def kernel(users, items, user_emb, item_emb, adj_src, adj_dst, adj_val,
           hp_src, hp_dst, hp_val, W, b):
    u_count, e = user_emb.shape
    n = u_count + item_emb.shape[0]
    npad = ((n + 127) // 128) * 128
    nlayers = W.shape[0]
    batch = users.shape[0]

    ego = jnp.concatenate([user_emb, item_emb], axis=0)
    ego = jnp.pad(ego, ((0, npad - n), (0, 0)))
    x2 = jnp.stack([ego[:, : e // 2], ego[:, e // 2:]], axis=0)

    nnz = adj_src.shape[0]
    quant = _NS * _C * _RPB * 2
    epad = ((nnz + quant - 1) // quant) * quant
    pad = epad - nnz
    a_src = jnp.pad(adj_src, (0, pad)).reshape(-1, _C)
    a_dst = jnp.pad(adj_dst, (0, pad)).reshape(-1, _C)
    a_val = jnp.pad(adj_val, (0, pad)).reshape(-1, _C)
    h_src = jnp.pad(hp_src, (0, pad)).reshape(-1, _C)
    h_dst = jnp.pad(hp_dst, (0, pad)).reshape(-1, _C)
    h_val = jnp.pad(hp_val, (0, pad)).reshape(-1, _C)

    spmm = _spmm_layer(npad, epad)
    x_flat = x2.reshape(2 * npad, 16)
    for k in range(nlayers):
        yp2, yd2 = _linear(x2, W[k])
        x_flat = spmm(yp2.reshape(2 * npad, 16), yd2.reshape(2 * npad, 16),
                      a_src, a_dst, a_val, h_src, h_dst, h_val, b[k])
        x2 = x_flat.reshape(2, npad, 16)

    uo, io = _gather_out(npad, u_count, batch)(x_flat, users, items)
    u2 = uo.reshape(2, batch, 16)
    i2 = io.reshape(2, batch, 16)
    user_embeddings = jnp.concatenate([u2[0], u2[1]], axis=1)
    item_embeddings = jnp.concatenate([i2[0], i2[1]], axis=1)
    return (user_embeddings, item_embeddings)
